# baseline (speedup 1.0000x reference)
import jax
import jax.numpy as jnp
from jax import lax
from jax.experimental import pallas as pl
from jax.experimental.pallas import tpu as pltpu
from jax.experimental.pallas import tpu_sc as plsc

N = 10000
NP = 10240
F = 128
E = 320000
NC, NS = 2, 16
NW = NC * NS
EPW = E // NW
CHUNK = 40
NCH = EPW // CHUNK
NBUF = 2
DRING = 5
CRING = 10
STRIPE = NP // NS
R = 1024
GRID = NP // R


def _fill_f32(ref, n, value):
    v = jnp.full((16,), value, dtype=jnp.float32)

    def body(i, _):
        ref[pl.ds(i * 16, 16)] = v
        return 0

    lax.fori_loop(0, n // 16, body, 0)


def _fill2_f32(ref, rows, value):
    v = jnp.full((16,), value, dtype=jnp.float32)

    def body(i, _):
        for j in range(8):
            ref[i, pl.ds(j * 16, 16)] = v
        return 0

    lax.fori_loop(0, rows, body, 0)


def _hist_body(col3_hbm, cnt_hbm, hist_sh, idxc_v, ones_v, tmp_v, *hsem):
    c = lax.axis_index("c")
    s = lax.axis_index("s")
    wid = c * NS + s

    pltpu.async_copy(col3_hbm.at[wid], idxc_v, hsem[0])
    _fill_f32(tmp_v, STRIPE, 0.0)
    pltpu.sync_copy(tmp_v, hist_sh.at[pl.ds(s * STRIPE, STRIPE)])
    _fill_f32(ones_v, CHUNK, 1.0)
    pltpu.make_async_copy(col3_hbm.at[wid], idxc_v, hsem[0]).wait()
    plsc.subcore_barrier()

    def group(k, _):
        for b in range(DRING):
            i = k * DRING + b

            @pl.when(k > 0)
            def _():
                pltpu.make_async_copy(ones_v, hist_sh.at[idxc_v.at[i]],
                                      hsem[b]).wait()

            pltpu.async_copy(ones_v, hist_sh.at[idxc_v.at[i]], hsem[b],
                             add=True)
        return 0

    lax.fori_loop(0, NCH // DRING, group, 0)
    for b in range(DRING):
        pltpu.make_async_copy(ones_v, hist_sh.at[idxc_v.at[0]], hsem[b]).wait()
    plsc.subcore_barrier()

    pltpu.sync_copy(hist_sh.at[pl.ds(s * STRIPE, STRIPE)], tmp_v)
    pltpu.sync_copy(tmp_v, cnt_hbm.at[c, pl.ds(s * STRIPE, STRIPE)])


_hist = pl.kernel(
    _hist_body,
    out_type=jax.ShapeDtypeStruct((NC, NP), jnp.float32),
    mesh=plsc.VectorSubcoreMesh(core_axis_name="c", subcore_axis_name="s"),
    scratch_types=[
        pltpu.VMEM_SHARED((NP,), jnp.float32),
        pltpu.VMEM((NCH, CHUNK), jnp.int32),
        pltpu.VMEM((CHUNK,), jnp.float32),
        pltpu.VMEM((STRIPE,), jnp.float32),
    ] + [pltpu.SemaphoreType.DMA] * DRING,
)


def _edge_body(row1_hbm, col1_hbm, h2_hbm, s_hbm, acc_sh, idxr_v, cidx_v,
               r0, r1, *sems):
    c = lax.axis_index("c")
    s = lax.axis_index("s")
    wid = c * NS + s
    rows = [r0, r1]
    gsem = sems[0:NBUF]
    csem = sems[NBUF:NBUF + CRING]
    zsem = sems[NBUF + CRING]
    bsem = sems[NBUF + CRING + 1]
    ebase = wid * EPW

    pltpu.async_copy(row1_hbm.at[pl.ds(ebase, EPW)], idxr_v, bsem)
    for d in range(CRING):
        pltpu.async_copy(col1_hbm.at[pl.ds(ebase + d * CHUNK, CHUNK)],
                         cidx_v.at[d], csem[d])
    _fill2_f32(rows[0], CHUNK, 0.0)
    for j in range(STRIPE // CHUNK):
        pltpu.async_copy(rows[0],
                         acc_sh.at[pl.ds(s * STRIPE + j * CHUNK, CHUNK)],
                         zsem)
    for j in range(STRIPE // CHUNK):
        pltpu.make_async_copy(
            rows[0], acc_sh.at[pl.ds(s * STRIPE + j * CHUNK, CHUNK)],
            zsem).wait()
    pltpu.make_async_copy(row1_hbm.at[pl.ds(ebase, EPW)], idxr_v,
                          bsem).wait()
    for b in range(NBUF):
        pltpu.async_copy(
            h2_hbm.at[idxr_v.at[pl.ds(b * CHUNK, CHUNK)]], rows[b], gsem[b])
    plsc.subcore_barrier()

    def group(k, _):
        for d in range(CRING):
            b = d % NBUF
            i = k * CRING + d
            pltpu.make_async_copy(
                col1_hbm.at[pl.ds(ebase, CHUNK)], cidx_v.at[d],
                csem[d]).wait()
            pltpu.make_async_copy(
                h2_hbm.at[idxr_v.at[pl.ds(i * CHUNK, CHUNK)]], rows[b],
                gsem[b]).wait()
            pltpu.sync_copy(rows[b], acc_sh.at[cidx_v.at[d]], add=True)

            @pl.when(i + NBUF < NCH)
            def _():
                pltpu.async_copy(
                    h2_hbm.at[idxr_v.at[pl.ds((i + NBUF) * CHUNK, CHUNK)]],
                    rows[b], gsem[b])

            @pl.when(i + CRING < NCH)
            def _():
                pltpu.async_copy(
                    col1_hbm.at[pl.ds(ebase + (i + CRING) * CHUNK, CHUNK)],
                    cidx_v.at[d], csem[d])
        return 0

    lax.fori_loop(0, NCH // CRING, group, 0)
    plsc.subcore_barrier()

    for j in range(STRIPE // CHUNK):
        r0b = s * STRIPE + j * CHUNK
        b = j % 2
        if j >= 2:
            prev = s * STRIPE + (j - 2) * CHUNK
            pltpu.make_async_copy(rows[b], s_hbm.at[c, pl.ds(prev, CHUNK)],
                                  gsem[b]).wait()
        pltpu.sync_copy(acc_sh.at[pl.ds(r0b, CHUNK)], rows[b])
        pltpu.async_copy(rows[b], s_hbm.at[c, pl.ds(r0b, CHUNK)], gsem[b])
    for j in range(STRIPE // CHUNK - 2, STRIPE // CHUNK):
        r0b = s * STRIPE + j * CHUNK
        pltpu.make_async_copy(rows[j % 2], s_hbm.at[c, pl.ds(r0b, CHUNK)],
                              gsem[j % 2]).wait()


_edges = pl.kernel(
    _edge_body,
    out_type=jax.ShapeDtypeStruct((NC, NP, F), jnp.float32),
    mesh=plsc.VectorSubcoreMesh(core_axis_name="c", subcore_axis_name="s"),
    scratch_types=[
        pltpu.VMEM_SHARED((NP, F), jnp.float32),
        pltpu.VMEM((EPW,), jnp.int32),
        pltpu.VMEM((CRING, CHUNK), jnp.int32),
    ] + [pltpu.VMEM((CHUNK, F), jnp.float32)] * NBUF
      + [pltpu.SemaphoreType.DMA] * (NBUF + CRING + 2),
)


def _eye128():
    a = lax.broadcasted_iota(jnp.int32, (F, F), 0)
    b = lax.broadcasted_iota(jnp.int32, (F, F), 1)
    return (a == b).astype(jnp.float32)


def _ln_mm_body(x_ref, g_ref, bt_ref, w_ref, cnt_ref, o_ref):
    xb = x_ref[...]
    mean = jnp.mean(xb, axis=1, keepdims=True)
    xc = xb - mean
    var = jnp.mean(xc * xc, axis=1, keepdims=True)
    h = xc * lax.rsqrt(var + 1e-5) * g_ref[...] + bt_ref[...]
    h = jnp.maximum(h, 0.0)
    hw = jnp.dot(h, w_ref[...], preferred_element_type=jnp.float32)
    cb = cnt_ref[...]
    dis = lax.rsqrt(cb[0] + cb[1] + 1.0)
    eye = _eye128()
    for r in range(R // F):
        diag = eye * dis[r][None, :]
        o_ref[r * F:(r + 1) * F, :] = jnp.dot(
            diag, hw[r * F:(r + 1) * F, :], preferred_element_type=jnp.float32)


def _final_body(s_ref, h2_ref, cnt_ref, b_ref, o_ref):
    sb = s_ref[...]
    t = sb[0] + sb[1] + h2_ref[...]
    cb = cnt_ref[...]
    dis = lax.rsqrt(cb[0] + cb[1] + 1.0)
    eye = _eye128()
    bias = b_ref[...]
    for r in range(R // F):
        diag = eye * dis[r][None, :]
        o_ref[r * F:(r + 1) * F, :] = jnp.dot(
            diag, t[r * F:(r + 1) * F, :], preferred_element_type=jnp.float32) + bias


@jax.jit
def kernel(x, edge_index, gamma, beta, W, b):
    edge_index = edge_index.astype(jnp.int32)
    col3 = edge_index[1].reshape(NW, NCH, CHUNK)
    cnt = _hist(col3)

    cnt3 = cnt.reshape(NC, NP // F, F)

    h2 = pl.pallas_call(
        _ln_mm_body,
        grid=(GRID,),
        in_specs=[
            pl.BlockSpec((R, F), lambda i: (i, 0)),
            pl.BlockSpec((1, F), lambda i: (0, 0)),
            pl.BlockSpec((1, F), lambda i: (0, 0)),
            pl.BlockSpec((F, F), lambda i: (0, 0)),
            pl.BlockSpec((NC, R // F, F), lambda i: (0, i, 0)),
        ],
        out_specs=pl.BlockSpec((R, F), lambda i: (i, 0)),
        out_shape=jax.ShapeDtypeStruct((NP, F), jnp.float32),
    )(x, gamma.reshape(1, F), beta.reshape(1, F), W, cnt3)

    s_part = _edges(edge_index[0], edge_index[1], h2)

    return pl.pallas_call(
        _final_body,
        grid=(GRID,),
        in_specs=[
            pl.BlockSpec((NC, R, F), lambda i: (0, i, 0)),
            pl.BlockSpec((R, F), lambda i: (i, 0)),
            pl.BlockSpec((NC, R // F, F), lambda i: (0, i, 0)),
            pl.BlockSpec((1, F), lambda i: (0, 0)),
        ],
        out_specs=pl.BlockSpec((R, F), lambda i: (i, 0)),
        out_shape=jax.ShapeDtypeStruct((N, F), jnp.float32),
    )(s_part, h2, cnt3, b.reshape(1, F))


# speedup vs baseline: 1.3864x; 1.3864x over previous
import jax
import jax.numpy as jnp
from jax import lax
from jax.experimental import pallas as pl
from jax.experimental.pallas import tpu as pltpu
from jax.experimental.pallas import tpu_sc as plsc

N = 10000
NP = 10240
F = 128
E = 320000
NC, NS = 2, 16
NW = NC * NS
CHUNK = 64
NCH = 162
EPW = NCH * CHUNK
EP = NW * EPW
NBUF = 3
DRING = 6
CRING = 6
STRIPE = NP // NS
R = 1024
GRID = NP // R


def _fill_f32(ref, n, value):
    v = jnp.full((16,), value, dtype=jnp.float32)

    def body(i, _):
        ref[pl.ds(i * 16, 16)] = v
        return 0

    lax.fori_loop(0, n // 16, body, 0)


def _fill2_f32(ref, rows, value):
    v = jnp.full((16,), value, dtype=jnp.float32)

    def body(i, _):
        for j in range(8):
            ref[i, pl.ds(j * 16, 16)] = v
        return 0

    lax.fori_loop(0, rows, body, 0)


def _hist_body(col3_hbm, cnt_hbm, hist_sh, idxc_v, ones_v, tmp_v, *hsem):
    c = lax.axis_index("c")
    s = lax.axis_index("s")
    wid = c * NS + s

    pltpu.async_copy(col3_hbm.at[wid], idxc_v, hsem[0])
    _fill_f32(tmp_v, STRIPE, 0.0)
    pltpu.sync_copy(tmp_v, hist_sh.at[pl.ds(s * STRIPE, STRIPE)])
    _fill_f32(ones_v, CHUNK, 1.0)
    pltpu.make_async_copy(col3_hbm.at[wid], idxc_v, hsem[0]).wait()
    plsc.subcore_barrier()

    def group(k, _):
        for b in range(DRING):
            i = k * DRING + b

            @pl.when(k > 0)
            def _():
                pltpu.make_async_copy(ones_v, hist_sh.at[idxc_v.at[i]],
                                      hsem[b]).wait()

            pltpu.async_copy(ones_v, hist_sh.at[idxc_v.at[i]], hsem[b],
                             add=True)
        return 0

    lax.fori_loop(0, NCH // DRING, group, 0)
    for b in range(DRING):
        pltpu.make_async_copy(ones_v, hist_sh.at[idxc_v.at[0]], hsem[b]).wait()
    plsc.subcore_barrier()

    pltpu.sync_copy(hist_sh.at[pl.ds(s * STRIPE, STRIPE)], tmp_v)
    pltpu.sync_copy(tmp_v, cnt_hbm.at[c, pl.ds(s * STRIPE, STRIPE)])


_hist = pl.kernel(
    _hist_body,
    out_type=jax.ShapeDtypeStruct((NC, NP), jnp.float32),
    mesh=plsc.VectorSubcoreMesh(core_axis_name="c", subcore_axis_name="s"),
    scratch_types=[
        pltpu.VMEM_SHARED((NP,), jnp.float32),
        pltpu.VMEM((NCH, CHUNK), jnp.int32),
        pltpu.VMEM((CHUNK,), jnp.float32),
        pltpu.VMEM((STRIPE,), jnp.float32),
    ] + [pltpu.SemaphoreType.DMA] * DRING,
)


def _edge_body(row1_hbm, col1_hbm, h2_hbm, s_hbm, acc_sh, idxr_v, cidx_v,
               r0, r1, r2, *sems):
    c = lax.axis_index("c")
    s = lax.axis_index("s")
    wid = c * NS + s
    rows = [r0, r1, r2]
    gsem = sems[0:NBUF]
    csem = sems[NBUF:NBUF + CRING]
    zsem = sems[NBUF + CRING]
    bsem = sems[NBUF + CRING + 1]
    ebase = wid * EPW

    pltpu.async_copy(row1_hbm.at[pl.ds(ebase, EPW)], idxr_v, bsem)
    for d in range(CRING):
        pltpu.async_copy(col1_hbm.at[pl.ds(ebase + d * CHUNK, CHUNK)],
                         cidx_v.at[d], csem[d])
    _fill2_f32(rows[0], CHUNK, 0.0)
    for j in range(STRIPE // CHUNK):
        pltpu.async_copy(rows[0],
                         acc_sh.at[pl.ds(s * STRIPE + j * CHUNK, CHUNK)],
                         zsem)
    for j in range(STRIPE // CHUNK):
        pltpu.make_async_copy(
            rows[0], acc_sh.at[pl.ds(s * STRIPE + j * CHUNK, CHUNK)],
            zsem).wait()
    pltpu.make_async_copy(row1_hbm.at[pl.ds(ebase, EPW)], idxr_v,
                          bsem).wait()
    for b in range(NBUF):
        pltpu.async_copy(
            h2_hbm.at[idxr_v.at[pl.ds(b * CHUNK, CHUNK)]], rows[b], gsem[b])
    plsc.subcore_barrier()

    def group(k, _):
        for d in range(CRING):
            b = d % NBUF
            i = k * CRING + d
            pltpu.make_async_copy(
                col1_hbm.at[pl.ds(ebase, CHUNK)], cidx_v.at[d],
                csem[d]).wait()
            pltpu.make_async_copy(
                h2_hbm.at[idxr_v.at[pl.ds(i * CHUNK, CHUNK)]], rows[b],
                gsem[b]).wait()
            pltpu.sync_copy(rows[b], acc_sh.at[cidx_v.at[d]], add=True)

            @pl.when(i + NBUF < NCH)
            def _():
                pltpu.async_copy(
                    h2_hbm.at[idxr_v.at[pl.ds((i + NBUF) * CHUNK, CHUNK)]],
                    rows[b], gsem[b])

            @pl.when(i + CRING < NCH)
            def _():
                pltpu.async_copy(
                    col1_hbm.at[pl.ds(ebase + (i + CRING) * CHUNK, CHUNK)],
                    cidx_v.at[d], csem[d])
        return 0

    lax.fori_loop(0, NCH // CRING, group, 0)
    plsc.subcore_barrier()

    for j in range(STRIPE // CHUNK):
        r0b = s * STRIPE + j * CHUNK
        b = j % 2
        if j >= 2:
            prev = s * STRIPE + (j - 2) * CHUNK
            pltpu.make_async_copy(rows[b], s_hbm.at[c, pl.ds(prev, CHUNK)],
                                  gsem[b]).wait()
        pltpu.sync_copy(acc_sh.at[pl.ds(r0b, CHUNK)], rows[b])
        pltpu.async_copy(rows[b], s_hbm.at[c, pl.ds(r0b, CHUNK)], gsem[b])
    for j in range(STRIPE // CHUNK - 2, STRIPE // CHUNK):
        r0b = s * STRIPE + j * CHUNK
        pltpu.make_async_copy(rows[j % 2], s_hbm.at[c, pl.ds(r0b, CHUNK)],
                              gsem[j % 2]).wait()


_edges = pl.kernel(
    _edge_body,
    out_type=jax.ShapeDtypeStruct((NC, NP, F), jnp.float32),
    mesh=plsc.VectorSubcoreMesh(core_axis_name="c", subcore_axis_name="s"),
    scratch_types=[
        pltpu.VMEM_SHARED((NP, F), jnp.float32),
        pltpu.VMEM((EPW,), jnp.int32),
        pltpu.VMEM((CRING, CHUNK), jnp.int32),
    ] + [pltpu.VMEM((CHUNK, F), jnp.float32)] * NBUF
      + [pltpu.SemaphoreType.DMA] * (NBUF + CRING + 2),
)


def _eye128():
    a = lax.broadcasted_iota(jnp.int32, (F, F), 0)
    b = lax.broadcasted_iota(jnp.int32, (F, F), 1)
    return (a == b).astype(jnp.float32)


def _ln_mm_body(x_ref, g_ref, bt_ref, w_ref, cnt_ref, o_ref):
    xb = x_ref[...]
    mean = jnp.mean(xb, axis=1, keepdims=True)
    xc = xb - mean
    var = jnp.mean(xc * xc, axis=1, keepdims=True)
    h = xc * lax.rsqrt(var + 1e-5) * g_ref[...] + bt_ref[...]
    h = jnp.maximum(h, 0.0)
    hw = jnp.dot(h, w_ref[...], preferred_element_type=jnp.float32)
    cb = cnt_ref[...]
    dis = lax.rsqrt(cb[0] + cb[1] + 1.0)
    eye = _eye128()
    for r in range(R // F):
        diag = eye * dis[r][None, :]
        o_ref[r * F:(r + 1) * F, :] = jnp.dot(
            diag, hw[r * F:(r + 1) * F, :], preferred_element_type=jnp.float32)


def _final_body(s_ref, h2_ref, cnt_ref, b_ref, o_ref):
    sb = s_ref[...]
    t = sb[0] + sb[1] + h2_ref[...]
    cb = cnt_ref[...]
    dis = lax.rsqrt(cb[0] + cb[1] + 1.0)
    eye = _eye128()
    bias = b_ref[...]
    for r in range(R // F):
        diag = eye * dis[r][None, :]
        o_ref[r * F:(r + 1) * F, :] = jnp.dot(
            diag, t[r * F:(r + 1) * F, :], preferred_element_type=jnp.float32) + bias


@jax.jit
def kernel(x, edge_index, gamma, beta, W, b):
    edge_index = edge_index.astype(jnp.int32)
    npad = EP - E
    ar = jnp.arange(npad, dtype=jnp.int32)
    row1 = jnp.concatenate([edge_index[0], ar & 8191])
    col1 = jnp.concatenate([edge_index[1], N + (ar & 127)])
    col3 = col1.reshape(NW, NCH, CHUNK)
    cnt = _hist(col3)

    cnt3 = cnt.reshape(NC, NP // F, F)

    h2 = pl.pallas_call(
        _ln_mm_body,
        grid=(GRID,),
        in_specs=[
            pl.BlockSpec((R, F), lambda i: (i, 0)),
            pl.BlockSpec((1, F), lambda i: (0, 0)),
            pl.BlockSpec((1, F), lambda i: (0, 0)),
            pl.BlockSpec((F, F), lambda i: (0, 0)),
            pl.BlockSpec((NC, R // F, F), lambda i: (0, i, 0)),
        ],
        out_specs=pl.BlockSpec((R, F), lambda i: (i, 0)),
        out_shape=jax.ShapeDtypeStruct((NP, F), jnp.float32),
    )(x, gamma.reshape(1, F), beta.reshape(1, F), W, cnt3)

    s_part = _edges(row1, col1, h2)

    return pl.pallas_call(
        _final_body,
        grid=(GRID,),
        in_specs=[
            pl.BlockSpec((NC, R, F), lambda i: (0, i, 0)),
            pl.BlockSpec((R, F), lambda i: (i, 0)),
            pl.BlockSpec((NC, R // F, F), lambda i: (0, i, 0)),
            pl.BlockSpec((1, F), lambda i: (0, 0)),
        ],
        out_specs=pl.BlockSpec((R, F), lambda i: (i, 0)),
        out_shape=jax.ShapeDtypeStruct((N, F), jnp.float32),
    )(s_part, h2, cnt3, b.reshape(1, F))


# speedup vs baseline: 1.4328x; 1.0334x over previous
import jax
import jax.numpy as jnp
from jax import lax
from jax.experimental import pallas as pl
from jax.experimental.pallas import tpu as pltpu
from jax.experimental.pallas import tpu_sc as plsc

N = 10000
NP = 10240
F = 128
E = 320000
NC, NS = 2, 16
NW = NC * NS
CHUNK = 64
NCH = 162
EPW = NCH * CHUNK
EP = NW * EPW
NBUF = 3
DRING = 6
CRING = 6
STRIPE = NP // NS
R = 2048
GRID = NP // R


def _fill_f32(ref, n, value):
    v = jnp.full((16,), value, dtype=jnp.float32)

    def body(i, _):
        ref[pl.ds(i * 16, 16)] = v
        return 0

    lax.fori_loop(0, n // 16, body, 0)


def _fill2_f32(ref, rows, value):
    v = jnp.full((16,), value, dtype=jnp.float32)

    def body(i, _):
        for j in range(8):
            ref[i, pl.ds(j * 16, 16)] = v
        return 0

    lax.fori_loop(0, rows, body, 0)


def _hist_body(col3_hbm, cnt_hbm, hist_sh, idxc_v, ones_v, tmp_v, *hsem):
    c = lax.axis_index("c")
    s = lax.axis_index("s")
    wid = c * NS + s

    pltpu.async_copy(col3_hbm.at[wid], idxc_v, hsem[0])
    _fill_f32(tmp_v, STRIPE, 0.0)
    pltpu.sync_copy(tmp_v, hist_sh.at[pl.ds(s * STRIPE, STRIPE)])
    _fill_f32(ones_v, CHUNK, 1.0)
    pltpu.make_async_copy(col3_hbm.at[wid], idxc_v, hsem[0]).wait()
    plsc.subcore_barrier()

    def group(k, _):
        for b in range(DRING):
            i = k * DRING + b

            @pl.when(k > 0)
            def _():
                pltpu.make_async_copy(ones_v, hist_sh.at[idxc_v.at[i]],
                                      hsem[b]).wait()

            pltpu.async_copy(ones_v, hist_sh.at[idxc_v.at[i]], hsem[b],
                             add=True)
        return 0

    lax.fori_loop(0, NCH // DRING, group, 0)
    for b in range(DRING):
        pltpu.make_async_copy(ones_v, hist_sh.at[idxc_v.at[0]], hsem[b]).wait()
    plsc.subcore_barrier()

    pltpu.sync_copy(hist_sh.at[pl.ds(s * STRIPE, STRIPE)], tmp_v)
    pltpu.sync_copy(tmp_v, cnt_hbm.at[c, pl.ds(s * STRIPE, STRIPE)])


_hist = pl.kernel(
    _hist_body,
    out_type=jax.ShapeDtypeStruct((NC, NP), jnp.float32),
    mesh=plsc.VectorSubcoreMesh(core_axis_name="c", subcore_axis_name="s"),
    scratch_types=[
        pltpu.VMEM_SHARED((NP,), jnp.float32),
        pltpu.VMEM((NCH, CHUNK), jnp.int32),
        pltpu.VMEM((CHUNK,), jnp.float32),
        pltpu.VMEM((STRIPE,), jnp.float32),
    ] + [pltpu.SemaphoreType.DMA] * DRING,
)


def _edge_body(row1_hbm, col1_hbm, h2_hbm, s_hbm, acc_sh, idxr_v, cidx_v,
               r0, r1, r2, *sems):
    c = lax.axis_index("c")
    s = lax.axis_index("s")
    wid = c * NS + s
    rows = [r0, r1, r2]
    gsem = sems[0:NBUF]
    csem = sems[NBUF:NBUF + CRING]
    zsem = sems[NBUF + CRING]
    bsem = sems[NBUF + CRING + 1]
    ebase = wid * EPW

    pltpu.async_copy(row1_hbm.at[pl.ds(ebase, EPW)], idxr_v, bsem)
    for d in range(CRING):
        pltpu.async_copy(col1_hbm.at[pl.ds(ebase + d * CHUNK, CHUNK)],
                         cidx_v.at[d], csem[d])
    _fill2_f32(rows[0], CHUNK, 0.0)
    for j in range(STRIPE // CHUNK):
        pltpu.async_copy(rows[0],
                         acc_sh.at[pl.ds(s * STRIPE + j * CHUNK, CHUNK)],
                         zsem)
    for j in range(STRIPE // CHUNK):
        pltpu.make_async_copy(
            rows[0], acc_sh.at[pl.ds(s * STRIPE + j * CHUNK, CHUNK)],
            zsem).wait()
    pltpu.make_async_copy(row1_hbm.at[pl.ds(ebase, EPW)], idxr_v,
                          bsem).wait()
    for b in range(NBUF):
        pltpu.async_copy(
            h2_hbm.at[idxr_v.at[pl.ds(b * CHUNK, CHUNK)]], rows[b], gsem[b])
    plsc.subcore_barrier()

    def group(k, _):
        for d in range(CRING):
            b = d % NBUF
            i = k * CRING + d
            pltpu.make_async_copy(
                col1_hbm.at[pl.ds(ebase, CHUNK)], cidx_v.at[d],
                csem[d]).wait()
            pltpu.make_async_copy(
                h2_hbm.at[idxr_v.at[pl.ds(i * CHUNK, CHUNK)]], rows[b],
                gsem[b]).wait()
            pltpu.sync_copy(rows[b], acc_sh.at[cidx_v.at[d]], add=True)

            @pl.when(i + NBUF < NCH)
            def _():
                pltpu.async_copy(
                    h2_hbm.at[idxr_v.at[pl.ds((i + NBUF) * CHUNK, CHUNK)]],
                    rows[b], gsem[b])

            @pl.when(i + CRING < NCH)
            def _():
                pltpu.async_copy(
                    col1_hbm.at[pl.ds(ebase + (i + CRING) * CHUNK, CHUNK)],
                    cidx_v.at[d], csem[d])
        return 0

    lax.fori_loop(0, NCH // CRING, group, 0)
    plsc.subcore_barrier()

    for j in range(STRIPE // CHUNK):
        r0b = s * STRIPE + j * CHUNK
        b = j % 2
        if j >= 2:
            prev = s * STRIPE + (j - 2) * CHUNK
            pltpu.make_async_copy(rows[b], s_hbm.at[c, pl.ds(prev, CHUNK)],
                                  gsem[b]).wait()
        pltpu.sync_copy(acc_sh.at[pl.ds(r0b, CHUNK)], rows[b])
        pltpu.async_copy(rows[b], s_hbm.at[c, pl.ds(r0b, CHUNK)], gsem[b])
    for j in range(STRIPE // CHUNK - 2, STRIPE // CHUNK):
        r0b = s * STRIPE + j * CHUNK
        pltpu.make_async_copy(rows[j % 2], s_hbm.at[c, pl.ds(r0b, CHUNK)],
                              gsem[j % 2]).wait()


_edges = pl.kernel(
    _edge_body,
    out_type=jax.ShapeDtypeStruct((NC, NP, F), jnp.float32),
    mesh=plsc.VectorSubcoreMesh(core_axis_name="c", subcore_axis_name="s"),
    scratch_types=[
        pltpu.VMEM_SHARED((NP, F), jnp.float32),
        pltpu.VMEM((EPW,), jnp.int32),
        pltpu.VMEM((CRING, CHUNK), jnp.int32),
    ] + [pltpu.VMEM((CHUNK, F), jnp.float32)] * NBUF
      + [pltpu.SemaphoreType.DMA] * (NBUF + CRING + 2),
)


def _eye128():
    a = lax.broadcasted_iota(jnp.int32, (F, F), 0)
    b = lax.broadcasted_iota(jnp.int32, (F, F), 1)
    return (a == b).astype(jnp.float32)


def _ln_mm_body(x_ref, g_ref, bt_ref, w_ref, cnt_ref, o_ref):
    xb = x_ref[...]
    mean = jnp.mean(xb, axis=1, keepdims=True)
    xc = xb - mean
    var = jnp.mean(xc * xc, axis=1, keepdims=True)
    h = xc * lax.rsqrt(var + 1e-5) * g_ref[...] + bt_ref[...]
    h = jnp.maximum(h, 0.0)
    hw = jnp.dot(h, w_ref[...], preferred_element_type=jnp.float32)
    cb = cnt_ref[...]
    dis = lax.rsqrt(cb[0] + cb[1] + 1.0)
    eye = _eye128()
    for r in range(R // F):
        diag = eye * dis[r][None, :]
        o_ref[r * F:(r + 1) * F, :] = jnp.dot(
            diag, hw[r * F:(r + 1) * F, :], preferred_element_type=jnp.float32)


def _final_body(s_ref, h2_ref, cnt_ref, b_ref, o_ref):
    sb = s_ref[...]
    t = sb[0] + sb[1] + h2_ref[...]
    cb = cnt_ref[...]
    dis = lax.rsqrt(cb[0] + cb[1] + 1.0)
    eye = _eye128()
    bias = b_ref[...]
    for r in range(R // F):
        diag = eye * dis[r][None, :]
        o_ref[r * F:(r + 1) * F, :] = jnp.dot(
            diag, t[r * F:(r + 1) * F, :], preferred_element_type=jnp.float32) + bias


@jax.jit
def kernel(x, edge_index, gamma, beta, W, b):
    edge_index = edge_index.astype(jnp.int32)
    npad = EP - E
    ar = jnp.arange(npad, dtype=jnp.int32)
    row1 = jnp.concatenate([edge_index[0], ar & 8191])
    col1 = jnp.concatenate([edge_index[1], N + (ar & 127)])
    col3 = col1.reshape(NW, NCH, CHUNK)
    cnt = _hist(col3)

    cnt3 = cnt.reshape(NC, NP // F, F)

    h2 = pl.pallas_call(
        _ln_mm_body,
        grid=(GRID,),
        in_specs=[
            pl.BlockSpec((R, F), lambda i: (i, 0)),
            pl.BlockSpec((1, F), lambda i: (0, 0)),
            pl.BlockSpec((1, F), lambda i: (0, 0)),
            pl.BlockSpec((F, F), lambda i: (0, 0)),
            pl.BlockSpec((NC, R // F, F), lambda i: (0, i, 0)),
        ],
        out_specs=pl.BlockSpec((R, F), lambda i: (i, 0)),
        out_shape=jax.ShapeDtypeStruct((NP, F), jnp.float32),
    )(x, gamma.reshape(1, F), beta.reshape(1, F), W, cnt3)

    s_part = _edges(row1, col1, h2)

    return pl.pallas_call(
        _final_body,
        grid=(GRID,),
        in_specs=[
            pl.BlockSpec((NC, R, F), lambda i: (0, i, 0)),
            pl.BlockSpec((R, F), lambda i: (i, 0)),
            pl.BlockSpec((NC, R // F, F), lambda i: (0, i, 0)),
            pl.BlockSpec((1, F), lambda i: (0, 0)),
        ],
        out_specs=pl.BlockSpec((R, F), lambda i: (i, 0)),
        out_shape=jax.ShapeDtypeStruct((N, F), jnp.float32),
    )(s_part, h2, cnt3, b.reshape(1, F))


# speedup vs baseline: 1.4554x; 1.0158x over previous
import jax
import jax.numpy as jnp
from jax import lax
from jax.experimental import pallas as pl
from jax.experimental.pallas import tpu as pltpu
from jax.experimental.pallas import tpu_sc as plsc

N = 10000
NP = 10240
F = 128
E = 320000
NC, NS = 2, 16
NW = NC * NS
CHUNK = 64
NCH = 162
EPW = NCH * CHUNK
EP = NW * EPW
NBUF = 3
DRING = 6
CRING = 6
STRIPE = NP // NS
R = 5120
GRID = NP // R


def _fill_f32(ref, n, value):
    v = jnp.full((16,), value, dtype=jnp.float32)

    def body(i, _):
        ref[pl.ds(i * 16, 16)] = v
        return 0

    lax.fori_loop(0, n // 16, body, 0)


def _fill2_f32(ref, rows, value):
    v = jnp.full((16,), value, dtype=jnp.float32)

    def body(i, _):
        for j in range(8):
            ref[i, pl.ds(j * 16, 16)] = v
        return 0

    lax.fori_loop(0, rows, body, 0)


def _hist_body(col3_hbm, cnt_hbm, hist_sh, idxc_v, ones_v, tmp_v, *hsem):
    c = lax.axis_index("c")
    s = lax.axis_index("s")
    wid = c * NS + s

    pltpu.async_copy(col3_hbm.at[wid], idxc_v, hsem[0])
    _fill_f32(tmp_v, STRIPE, 0.0)
    pltpu.sync_copy(tmp_v, hist_sh.at[pl.ds(s * STRIPE, STRIPE)])
    _fill_f32(ones_v, CHUNK, 1.0)
    pltpu.make_async_copy(col3_hbm.at[wid], idxc_v, hsem[0]).wait()
    plsc.subcore_barrier()

    def group(k, _):
        for b in range(DRING):
            i = k * DRING + b

            @pl.when(k > 0)
            def _():
                pltpu.make_async_copy(ones_v, hist_sh.at[idxc_v.at[i]],
                                      hsem[b]).wait()

            pltpu.async_copy(ones_v, hist_sh.at[idxc_v.at[i]], hsem[b],
                             add=True)
        return 0

    lax.fori_loop(0, NCH // DRING, group, 0)
    for b in range(DRING):
        pltpu.make_async_copy(ones_v, hist_sh.at[idxc_v.at[0]], hsem[b]).wait()
    plsc.subcore_barrier()

    pltpu.sync_copy(hist_sh.at[pl.ds(s * STRIPE, STRIPE)], tmp_v)
    pltpu.sync_copy(tmp_v, cnt_hbm.at[c, pl.ds(s * STRIPE, STRIPE)])


_hist = pl.kernel(
    _hist_body,
    out_type=jax.ShapeDtypeStruct((NC, NP), jnp.float32),
    mesh=plsc.VectorSubcoreMesh(core_axis_name="c", subcore_axis_name="s"),
    scratch_types=[
        pltpu.VMEM_SHARED((NP,), jnp.float32),
        pltpu.VMEM((NCH, CHUNK), jnp.int32),
        pltpu.VMEM((CHUNK,), jnp.float32),
        pltpu.VMEM((STRIPE,), jnp.float32),
    ] + [pltpu.SemaphoreType.DMA] * DRING,
)


def _edge_body(row1_hbm, col1_hbm, h2_hbm, s_hbm, acc_sh, idxr_v, cidx_v,
               r0, r1, r2, *sems):
    c = lax.axis_index("c")
    s = lax.axis_index("s")
    wid = c * NS + s
    rows = [r0, r1, r2]
    gsem = sems[0:NBUF]
    csem = sems[NBUF:NBUF + CRING]
    zsem = sems[NBUF + CRING]
    bsem = sems[NBUF + CRING + 1]
    ebase = wid * EPW

    pltpu.async_copy(row1_hbm.at[pl.ds(ebase, EPW)], idxr_v, bsem)
    for d in range(CRING):
        pltpu.async_copy(col1_hbm.at[pl.ds(ebase + d * CHUNK, CHUNK)],
                         cidx_v.at[d], csem[d])
    _fill2_f32(rows[0], CHUNK, 0.0)
    for j in range(STRIPE // CHUNK):
        pltpu.async_copy(rows[0],
                         acc_sh.at[pl.ds(s * STRIPE + j * CHUNK, CHUNK)],
                         zsem)
    for j in range(STRIPE // CHUNK):
        pltpu.make_async_copy(
            rows[0], acc_sh.at[pl.ds(s * STRIPE + j * CHUNK, CHUNK)],
            zsem).wait()
    pltpu.make_async_copy(row1_hbm.at[pl.ds(ebase, EPW)], idxr_v,
                          bsem).wait()
    for b in range(NBUF):
        pltpu.async_copy(
            h2_hbm.at[idxr_v.at[pl.ds(b * CHUNK, CHUNK)]], rows[b], gsem[b])
    plsc.subcore_barrier()

    def group(k, _):
        for d in range(CRING):
            b = d % NBUF
            i = k * CRING + d
            pltpu.make_async_copy(
                col1_hbm.at[pl.ds(ebase, CHUNK)], cidx_v.at[d],
                csem[d]).wait()
            pltpu.make_async_copy(
                h2_hbm.at[idxr_v.at[pl.ds(i * CHUNK, CHUNK)]], rows[b],
                gsem[b]).wait()
            pltpu.sync_copy(rows[b], acc_sh.at[cidx_v.at[d]], add=True)

            @pl.when(i + NBUF < NCH)
            def _():
                pltpu.async_copy(
                    h2_hbm.at[idxr_v.at[pl.ds((i + NBUF) * CHUNK, CHUNK)]],
                    rows[b], gsem[b])

            @pl.when(i + CRING < NCH)
            def _():
                pltpu.async_copy(
                    col1_hbm.at[pl.ds(ebase + (i + CRING) * CHUNK, CHUNK)],
                    cidx_v.at[d], csem[d])
        return 0

    lax.fori_loop(0, NCH // CRING, group, 0)
    plsc.subcore_barrier()

    for j in range(STRIPE // CHUNK):
        r0b = s * STRIPE + j * CHUNK
        b = j % 2
        if j >= 2:
            prev = s * STRIPE + (j - 2) * CHUNK
            pltpu.make_async_copy(rows[b], s_hbm.at[c, pl.ds(prev, CHUNK)],
                                  gsem[b]).wait()
        pltpu.sync_copy(acc_sh.at[pl.ds(r0b, CHUNK)], rows[b])
        pltpu.async_copy(rows[b], s_hbm.at[c, pl.ds(r0b, CHUNK)], gsem[b])
    for j in range(STRIPE // CHUNK - 2, STRIPE // CHUNK):
        r0b = s * STRIPE + j * CHUNK
        pltpu.make_async_copy(rows[j % 2], s_hbm.at[c, pl.ds(r0b, CHUNK)],
                              gsem[j % 2]).wait()


_edges = pl.kernel(
    _edge_body,
    out_type=jax.ShapeDtypeStruct((NC, NP, F), jnp.float32),
    mesh=plsc.VectorSubcoreMesh(core_axis_name="c", subcore_axis_name="s"),
    scratch_types=[
        pltpu.VMEM_SHARED((NP, F), jnp.float32),
        pltpu.VMEM((EPW,), jnp.int32),
        pltpu.VMEM((CRING, CHUNK), jnp.int32),
    ] + [pltpu.VMEM((CHUNK, F), jnp.float32)] * NBUF
      + [pltpu.SemaphoreType.DMA] * (NBUF + CRING + 2),
)


def _eye128():
    a = lax.broadcasted_iota(jnp.int32, (F, F), 0)
    b = lax.broadcasted_iota(jnp.int32, (F, F), 1)
    return (a == b).astype(jnp.float32)


def _ln_mm_body(x_ref, g_ref, bt_ref, w_ref, cnt_ref, o_ref):
    xb = x_ref[...]
    mean = jnp.mean(xb, axis=1, keepdims=True)
    xc = xb - mean
    var = jnp.mean(xc * xc, axis=1, keepdims=True)
    h = xc * lax.rsqrt(var + 1e-5) * g_ref[...] + bt_ref[...]
    h = jnp.maximum(h, 0.0)
    hw = jnp.dot(h, w_ref[...], preferred_element_type=jnp.float32)
    cb = cnt_ref[...]
    dis = lax.rsqrt(cb[0] + cb[1] + 1.0)
    eye = _eye128()
    for r in range(R // F):
        diag = eye * dis[r][None, :]
        o_ref[r * F:(r + 1) * F, :] = jnp.dot(
            diag, hw[r * F:(r + 1) * F, :], preferred_element_type=jnp.float32)


def _final_body(s_ref, h2_ref, cnt_ref, b_ref, o_ref):
    sb = s_ref[...]
    t = sb[0] + sb[1] + h2_ref[...]
    cb = cnt_ref[...]
    dis = lax.rsqrt(cb[0] + cb[1] + 1.0)
    eye = _eye128()
    bias = b_ref[...]
    for r in range(R // F):
        diag = eye * dis[r][None, :]
        o_ref[r * F:(r + 1) * F, :] = jnp.dot(
            diag, t[r * F:(r + 1) * F, :], preferred_element_type=jnp.float32) + bias


@jax.jit
def kernel(x, edge_index, gamma, beta, W, b):
    edge_index = edge_index.astype(jnp.int32)
    npad = EP - E
    ar = jnp.arange(npad, dtype=jnp.int32)
    row1 = jnp.concatenate([edge_index[0], ar & 8191])
    col1 = jnp.concatenate([edge_index[1], N + (ar & 127)])
    col3 = col1.reshape(NW, NCH, CHUNK)
    cnt = _hist(col3)

    cnt3 = cnt.reshape(NC, NP // F, F)

    h2 = pl.pallas_call(
        _ln_mm_body,
        grid=(GRID,),
        in_specs=[
            pl.BlockSpec((R, F), lambda i: (i, 0)),
            pl.BlockSpec((1, F), lambda i: (0, 0)),
            pl.BlockSpec((1, F), lambda i: (0, 0)),
            pl.BlockSpec((F, F), lambda i: (0, 0)),
            pl.BlockSpec((NC, R // F, F), lambda i: (0, i, 0)),
        ],
        out_specs=pl.BlockSpec((R, F), lambda i: (i, 0)),
        out_shape=jax.ShapeDtypeStruct((NP, F), jnp.float32),
    )(x, gamma.reshape(1, F), beta.reshape(1, F), W, cnt3)

    s_part = _edges(row1, col1, h2)

    return pl.pallas_call(
        _final_body,
        grid=(GRID,),
        in_specs=[
            pl.BlockSpec((NC, R, F), lambda i: (0, i, 0)),
            pl.BlockSpec((R, F), lambda i: (i, 0)),
            pl.BlockSpec((NC, R // F, F), lambda i: (0, i, 0)),
            pl.BlockSpec((1, F), lambda i: (0, 0)),
        ],
        out_specs=pl.BlockSpec((R, F), lambda i: (i, 0)),
        out_shape=jax.ShapeDtypeStruct((N, F), jnp.float32),
    )(s_part, h2, cnt3, b.reshape(1, F))
